# Initial kernel scaffold; baseline (speedup 1.0000x reference)
import jax
import jax.numpy as jnp
from jax.experimental import pallas as pl


def kernel(x, edge_index, W1, b1, W2, b2, W3, b3):
    raise NotImplementedError("write your pallas kernel here")


# speedup vs baseline: 11.2867x; 11.2867x over previous
import functools

import jax
import jax.numpy as jnp
from jax import lax
from jax.experimental import pallas as pl
from jax.experimental.pallas import tpu as pltpu
from jax.experimental.pallas import tpu_sc as plsc

N = 10000
E = 320000
D = 128

NC = 2
NS = 16
NW = NC * NS
EPW = E // NW
C = 80
NCHUNK = EPW // C
NP = 10240
RPS = NP // NS
ZB = 32

_mesh = plsc.VectorSubcoreMesh(
    core_axis_name="c", subcore_axis_name="s", num_cores=NC, num_subcores=NS)


@functools.partial(
    pl.kernel,
    out_type=jax.ShapeDtypeStruct((NC, NP, 16), jnp.float32),
    mesh=_mesh,
    scratch_types=[
        pltpu.VMEM((C,), jnp.int32),
        pltpu.VMEM((C, 16), jnp.float32),
        pltpu.VMEM((ZB, 16), jnp.float32),
        pltpu.VMEM_SHARED((NP, 16), jnp.float32),
    ],
)
def _sc_degree(dst_hbm, out_hbm, didx, ones_v, zb, deg_sh):
    cid = lax.axis_index("c")
    sid = lax.axis_index("s")
    wid = sid * NC + cid
    zero = jnp.zeros((16,), jnp.float32)
    one = jnp.ones((16,), jnp.float32)
    for i in range(ZB):
        zb[i, :] = zero
    for i in range(C):
        ones_v[i, :] = one
    row0 = sid * RPS
    for t in range(RPS // ZB):
        pltpu.sync_copy(zb, deg_sh.at[pl.ds(row0 + t * ZB, ZB)])
    plsc.subcore_barrier()

    def body(k, carry):
        base = pl.multiple_of(wid * EPW + k * C, 8)
        pltpu.sync_copy(dst_hbm.at[pl.ds(base, C)], didx)
        pltpu.sync_copy(ones_v, deg_sh.at[didx], add=True)
        return carry

    lax.fori_loop(0, NCHUNK, body, 0)
    plsc.subcore_barrier()
    pltpu.sync_copy(deg_sh.at[pl.ds(row0, RPS)],
                    out_hbm.at[cid, pl.ds(row0, RPS)])


@functools.partial(
    pl.kernel,
    out_type=jax.ShapeDtypeStruct((NC, NP, D), jnp.float32),
    mesh=_mesh,
    scratch_types=[
        pltpu.VMEM((C,), jnp.int32),
        pltpu.VMEM((C,), jnp.int32),
        pltpu.VMEM((C, D), jnp.float32),
        pltpu.VMEM((ZB, D), jnp.float32),
        pltpu.VMEM_SHARED((NP, D), jnp.float32),
        pltpu.SemaphoreType.DMA,
    ],
)
def _sc_scatter(y_hbm, src_hbm, dst_hbm, out_hbm,
                sidx, didx, rows, zb, z_sh, sem):
    cid = lax.axis_index("c")
    sid = lax.axis_index("s")
    wid = sid * NC + cid
    zero = jnp.zeros((16,), jnp.float32)
    for i in range(ZB):
        for j in range(D // 16):
            zb[i, pl.ds(j * 16, 16)] = zero
    row0 = sid * RPS
    for t in range(RPS // ZB):
        pltpu.sync_copy(zb, z_sh.at[pl.ds(row0 + t * ZB, ZB)])
    plsc.subcore_barrier()

    def body(k, carry):
        base = pl.multiple_of(wid * EPW + k * C, 8)
        pltpu.sync_copy(src_hbm.at[pl.ds(base, C)], sidx)
        pltpu.sync_copy(dst_hbm.at[pl.ds(base, C)], didx)
        pltpu.async_copy(y_hbm.at[sidx], rows, sem).wait()
        pltpu.sync_copy(rows, z_sh.at[didx], add=True)
        return carry

    lax.fori_loop(0, NCHUNK, body, 0)
    plsc.subcore_barrier()
    pltpu.sync_copy(z_sh.at[pl.ds(row0, RPS)],
                    out_hbm.at[cid, pl.ds(row0, RPS)])


BN = 1000


def _tc_first_body(degp_ref, x_ref, w_ref, dinv_ref, y_ref):
    deg = degp_ref[0, :, 0:1] + degp_ref[1, :, 0:1] + 1.0
    dinv = jnp.broadcast_to(lax.rsqrt(deg), (BN, D))
    dinv_ref[...] = dinv
    y_ref[...] = jnp.dot(x_ref[...], w_ref[...],
                         preferred_element_type=jnp.float32) * dinv


def _tc_mid_body(zp_ref, y_ref, dinv_ref, b_ref, w_ref, out_ref):
    dinv = dinv_ref[...]
    h = zp_ref[0] + zp_ref[1] + y_ref[...]
    h = jax.nn.relu(dinv * h + b_ref[...])
    out_ref[...] = jnp.dot(h, w_ref[...],
                           preferred_element_type=jnp.float32) * dinv


def _tc_last_body(zp_ref, y_ref, dinv_ref, b_ref, out_ref):
    dinv = dinv_ref[...]
    out_ref[...] = dinv * (zp_ref[0] + zp_ref[1] + y_ref[...]) + b_ref[...]


def _row_spec(shape3=False):
    if shape3:
        return pl.BlockSpec((NC, BN, D), lambda i: (0, i, 0))
    return pl.BlockSpec((BN, D), lambda i: (i, 0))


_full_w = pl.BlockSpec((D, D), lambda i: (0, 0))
_full_b = pl.BlockSpec((1, D), lambda i: (0, 0))

_tc_first = pl.pallas_call(
    _tc_first_body,
    grid=(N // BN,),
    in_specs=[pl.BlockSpec((NC, BN, 16), lambda i: (0, i, 0)),
              _row_spec(), _full_w],
    out_specs=[_row_spec(), _row_spec()],
    out_shape=[jax.ShapeDtypeStruct((N, D), jnp.float32),
               jax.ShapeDtypeStruct((N, D), jnp.float32)],
)

_tc_mid = pl.pallas_call(
    _tc_mid_body,
    grid=(N // BN,),
    in_specs=[_row_spec(True), _row_spec(), _row_spec(), _full_b, _full_w],
    out_specs=_row_spec(),
    out_shape=jax.ShapeDtypeStruct((N, D), jnp.float32),
)

_tc_last = pl.pallas_call(
    _tc_last_body,
    grid=(N // BN,),
    in_specs=[_row_spec(True), _row_spec(), _row_spec(), _full_b],
    out_specs=_row_spec(),
    out_shape=jax.ShapeDtypeStruct((N, D), jnp.float32),
)


def kernel(x, edge_index, W1, b1, W2, b2, W3, b3):
    src = edge_index[0].astype(jnp.int32)
    dst = edge_index[1].astype(jnp.int32)
    degp = _sc_degree(dst)
    dinv_b, y = _tc_first(degp, x, W1)
    zp = _sc_scatter(y, src, dst)
    y = _tc_mid(zp, y, dinv_b, b1.reshape(1, D), W2)
    zp = _sc_scatter(y, src, dst)
    y = _tc_mid(zp, y, dinv_b, b2.reshape(1, D), W3)
    zp = _sc_scatter(y, src, dst)
    return _tc_last(zp, y, dinv_b, b3.reshape(1, D))


# speedup vs baseline: 16.7542x; 1.4844x over previous
import functools

import jax
import jax.numpy as jnp
from jax import lax
from jax.experimental import pallas as pl
from jax.experimental.pallas import tpu as pltpu
from jax.experimental.pallas import tpu_sc as plsc

N = 10000
E = 320000
D = 128

NC = 2
NS = 16
NW = NC * NS
EPW = E // NW
C = 80
NCHUNK = EPW // C
NP = 10240
RPS = NP // NS
ZB = 32

_mesh = plsc.VectorSubcoreMesh(
    core_axis_name="c", subcore_axis_name="s", num_cores=NC, num_subcores=NS)


@functools.partial(
    pl.kernel,
    out_type=jax.ShapeDtypeStruct((NC, NP, 16), jnp.float32),
    mesh=_mesh,
    scratch_types=[
        pltpu.VMEM((C,), jnp.int32),
        pltpu.VMEM((C, 16), jnp.float32),
        pltpu.VMEM((ZB, 16), jnp.float32),
        pltpu.VMEM_SHARED((NP, 16), jnp.float32),
    ],
)
def _sc_degree(dst_hbm, out_hbm, didx, ones_v, zb, deg_sh):
    cid = lax.axis_index("c")
    sid = lax.axis_index("s")
    wid = sid * NC + cid
    zero = jnp.zeros((16,), jnp.float32)
    one = jnp.ones((16,), jnp.float32)
    for i in range(ZB):
        zb[i, :] = zero
    for i in range(C):
        ones_v[i, :] = one
    row0 = sid * RPS
    for t in range(RPS // ZB):
        pltpu.sync_copy(zb, deg_sh.at[pl.ds(row0 + t * ZB, ZB)])
    plsc.subcore_barrier()

    def body(k, carry):
        base = pl.multiple_of(wid * EPW + k * C, 8)
        pltpu.sync_copy(dst_hbm.at[pl.ds(base, C)], didx)
        pltpu.sync_copy(ones_v, deg_sh.at[didx], add=True)
        return carry

    lax.fori_loop(0, NCHUNK, body, 0)
    plsc.subcore_barrier()
    pltpu.sync_copy(deg_sh.at[pl.ds(row0, RPS)],
                    out_hbm.at[cid, pl.ds(row0, RPS)])


NB = 3
NGROUP = NCHUNK // NB


_SC_SCATTER_KW = dict(
    out_type=jax.ShapeDtypeStruct((NC, NP, D), jnp.float32),
    mesh=_mesh,
    scratch_types=[
        [pltpu.VMEM((C,), jnp.int32)] * NB,
        pltpu.VMEM((C,), jnp.int32),
        [pltpu.VMEM((C, D), jnp.float32)] * NB,
        pltpu.VMEM((ZB, D), jnp.float32),
        pltpu.VMEM_SHARED((NP, D), jnp.float32),
        [pltpu.SemaphoreType.DMA] * NB,
    ],
)


def _sc_scatter_body(y_hbm, src_hbm, dst_hbm, out_hbm,
                     sidx, didx, rows, zb, z_sh, sems):
    cid = lax.axis_index("c")
    sid = lax.axis_index("s")
    wid = sid * NC + cid
    zero = jnp.zeros((16,), jnp.float32)
    for i in range(ZB):
        for j in range(D // 16):
            zb[i, pl.ds(j * 16, 16)] = zero
    row0 = sid * RPS
    for t in range(RPS // ZB):
        pltpu.sync_copy(zb, z_sh.at[pl.ds(row0 + t * ZB, ZB)])
    plsc.subcore_barrier()

    def gather(k, b):
        base = pl.multiple_of(wid * EPW + k * C, 8)
        pltpu.sync_copy(src_hbm.at[pl.ds(base, C)], sidx[b])
        return pltpu.async_copy(y_hbm.at[sidx[b]], rows[b], sems[b])

    def scatter(k, b):
        base = pl.multiple_of(wid * EPW + k * C, 8)
        pltpu.sync_copy(dst_hbm.at[pl.ds(base, C)], didx)
        pltpu.sync_copy(rows[b], z_sh.at[didx], add=True)

    def run(k0, nb):
        descs = [gather(k0 + b, b) for b in range(nb)]
        for b in range(nb):
            descs[b].wait()
            scatter(k0 + b, b)

    def group(g, carry):
        run(g * NB, NB)
        return carry

    lax.fori_loop(0, NGROUP, group, 0)
    run(NGROUP * NB, NCHUNK - NGROUP * NB)

    plsc.subcore_barrier()
    pltpu.sync_copy(z_sh.at[pl.ds(row0, RPS)],
                    out_hbm.at[cid, pl.ds(row0, RPS)])


_sc_scatter = pl.kernel(_sc_scatter_body, **_SC_SCATTER_KW)


BN = 1000


def _tc_first_body(degp_ref, x_ref, w_ref, dinv_ref, y_ref):
    deg = degp_ref[0, :, 0:1] + degp_ref[1, :, 0:1] + 1.0
    dinv = jnp.broadcast_to(lax.rsqrt(deg), (BN, D))
    dinv_ref[...] = dinv
    y_ref[...] = jnp.dot(x_ref[...], w_ref[...],
                         preferred_element_type=jnp.float32) * dinv


def _tc_mid_body(zp_ref, y_ref, dinv_ref, b_ref, w_ref, out_ref):
    dinv = dinv_ref[...]
    h = zp_ref[0] + zp_ref[1] + y_ref[...]
    h = jax.nn.relu(dinv * h + b_ref[...])
    out_ref[...] = jnp.dot(h, w_ref[...],
                           preferred_element_type=jnp.float32) * dinv


def _tc_last_body(zp_ref, y_ref, dinv_ref, b_ref, out_ref):
    dinv = dinv_ref[...]
    out_ref[...] = dinv * (zp_ref[0] + zp_ref[1] + y_ref[...]) + b_ref[...]


def _row_spec(shape3=False):
    if shape3:
        return pl.BlockSpec((NC, BN, D), lambda i: (0, i, 0))
    return pl.BlockSpec((BN, D), lambda i: (i, 0))


_full_w = pl.BlockSpec((D, D), lambda i: (0, 0))
_full_b = pl.BlockSpec((1, D), lambda i: (0, 0))

_tc_first = pl.pallas_call(
    _tc_first_body,
    grid=(N // BN,),
    in_specs=[pl.BlockSpec((NC, BN, 16), lambda i: (0, i, 0)),
              _row_spec(), _full_w],
    out_specs=[_row_spec(), _row_spec()],
    out_shape=[jax.ShapeDtypeStruct((N, D), jnp.float32),
               jax.ShapeDtypeStruct((N, D), jnp.float32)],
)

_tc_mid = pl.pallas_call(
    _tc_mid_body,
    grid=(N // BN,),
    in_specs=[_row_spec(True), _row_spec(), _row_spec(), _full_b, _full_w],
    out_specs=_row_spec(),
    out_shape=jax.ShapeDtypeStruct((N, D), jnp.float32),
)

_tc_last = pl.pallas_call(
    _tc_last_body,
    grid=(N // BN,),
    in_specs=[_row_spec(True), _row_spec(), _row_spec(), _full_b],
    out_specs=_row_spec(),
    out_shape=jax.ShapeDtypeStruct((N, D), jnp.float32),
)


def kernel(x, edge_index, W1, b1, W2, b2, W3, b3):
    src = edge_index[0].astype(jnp.int32)
    dst = edge_index[1].astype(jnp.int32)
    degp = _sc_degree(dst)
    dinv_b, y = _tc_first(degp, x, W1)
    zp = _sc_scatter(y, src, dst)
    y = _tc_mid(zp, y, dinv_b, b1.reshape(1, D), W2)
    zp = _sc_scatter(y, src, dst)
    y = _tc_mid(zp, y, dinv_b, b2.reshape(1, D), W3)
    zp = _sc_scatter(y, src, dst)
    return _tc_last(zp, y, dinv_b, b3.reshape(1, D))


# speedup vs baseline: 18.4678x; 1.1023x over previous
import functools

import jax
import jax.numpy as jnp
from jax import lax
from jax.experimental import pallas as pl
from jax.experimental.pallas import tpu as pltpu
from jax.experimental.pallas import tpu_sc as plsc

N = 10000
E = 320000
D = 128

NC = 2
NS = 16
NW = NC * NS
EPW = E // NW
C = 80
NCHUNK = EPW // C
NP = 10240
RPS = NP // NS
ZB = 32

_mesh = plsc.VectorSubcoreMesh(
    core_axis_name="c", subcore_axis_name="s", num_cores=NC, num_subcores=NS)


@functools.partial(
    pl.kernel,
    out_type=jax.ShapeDtypeStruct((NC, NP, 16), jnp.float32),
    mesh=_mesh,
    scratch_types=[
        [pltpu.VMEM((C,), jnp.int32)] * 3,
        pltpu.VMEM((C, 16), jnp.float32),
        pltpu.VMEM((ZB, 16), jnp.float32),
        pltpu.VMEM_SHARED((NP, 16), jnp.float32),
        [pltpu.SemaphoreType.DMA] * 3,
    ],
)
def _sc_degree(dst_hbm, out_hbm, didx, ones_v, zb, deg_sh, dsems):
    cid = lax.axis_index("c")
    sid = lax.axis_index("s")
    wid = sid * NC + cid
    zero = jnp.zeros((16,), jnp.float32)
    one = jnp.ones((16,), jnp.float32)
    for i in range(ZB):
        zb[i, :] = zero
    for i in range(C):
        ones_v[i, :] = one
    row0 = sid * RPS
    for t in range(RPS // ZB):
        pltpu.sync_copy(zb, deg_sh.at[pl.ds(row0 + t * ZB, ZB)])
    plsc.subcore_barrier()

    def body(k, carry):
        base = pl.multiple_of(wid * EPW + k * C, 8)
        pltpu.sync_copy(dst_hbm.at[pl.ds(base, C)], didx[0])
        pltpu.sync_copy(ones_v, deg_sh.at[didx[0]], add=True)
        return carry

    lax.fori_loop(0, NCHUNK, body, 0)
    plsc.subcore_barrier()
    pltpu.sync_copy(deg_sh.at[pl.ds(row0, RPS)],
                    out_hbm.at[cid, pl.ds(row0, RPS)])


NB = 3
NGROUP = NCHUNK // NB


_SC_SCATTER_KW = dict(
    out_type=jax.ShapeDtypeStruct((NC, NP, D), jnp.float32),
    mesh=_mesh,
    scratch_types=[
        [pltpu.VMEM((C,), jnp.int32)] * NB,
        [pltpu.VMEM((C,), jnp.int32)] * NB,
        [pltpu.VMEM((C, D), jnp.float32)] * NB,
        pltpu.VMEM((ZB, D), jnp.float32),
        pltpu.VMEM_SHARED((NP, D), jnp.float32),
        [pltpu.SemaphoreType.DMA] * NB,
        [pltpu.SemaphoreType.DMA] * NB,
        [pltpu.SemaphoreType.DMA] * NB,
    ],
)


def _sc_scatter_body(y_hbm, src_hbm, dst_hbm, out_hbm,
                     sidx, didx, rows, zb, z_sh, ssems, dsems, gsems):
    cid = lax.axis_index("c")
    sid = lax.axis_index("s")
    wid = sid * NC + cid
    zero = jnp.zeros((16,), jnp.float32)
    for i in range(ZB):
        for j in range(D // 16):
            zb[i, pl.ds(j * 16, 16)] = zero
    row0 = sid * RPS
    for t in range(RPS // ZB):
        pltpu.sync_copy(zb, z_sh.at[pl.ds(row0 + t * ZB, ZB)])
    plsc.subcore_barrier()

    def run(k0, nb):
        ds, dd = [], []
        for b in range(nb):
            base = pl.multiple_of(wid * EPW + (k0 + b) * C, 8)
            ds.append(pltpu.async_copy(src_hbm.at[pl.ds(base, C)],
                                       sidx[b], ssems[b]))
            dd.append(pltpu.async_copy(dst_hbm.at[pl.ds(base, C)],
                                       didx[b], dsems[b]))
        dg = []
        for b in range(nb):
            ds[b].wait()
            dg.append(pltpu.async_copy(y_hbm.at[sidx[b]],
                                       rows[b], gsems[b]))
        for b in range(nb):
            dg[b].wait()
            dd[b].wait()
            pltpu.sync_copy(rows[b], z_sh.at[didx[b]], add=True)

    def group(g, carry):
        run(g * NB, NB)
        return carry

    lax.fori_loop(0, NGROUP, group, 0)
    run(NGROUP * NB, NCHUNK - NGROUP * NB)

    plsc.subcore_barrier()
    pltpu.sync_copy(z_sh.at[pl.ds(row0, RPS)],
                    out_hbm.at[cid, pl.ds(row0, RPS)])


_sc_scatter = pl.kernel(_sc_scatter_body, **_SC_SCATTER_KW)


BN = 1000


def _tc_first_body(degp_ref, x_ref, w_ref, dinv_ref, y_ref):
    deg = degp_ref[0, :, 0:1] + degp_ref[1, :, 0:1] + 1.0
    dinv = jnp.broadcast_to(lax.rsqrt(deg), (BN, D))
    dinv_ref[...] = dinv
    y_ref[...] = jnp.dot(x_ref[...], w_ref[...],
                         preferred_element_type=jnp.float32) * dinv


def _tc_mid_body(zp_ref, y_ref, dinv_ref, b_ref, w_ref, out_ref):
    dinv = dinv_ref[...]
    h = zp_ref[0] + zp_ref[1] + y_ref[...]
    h = jax.nn.relu(dinv * h + b_ref[...])
    out_ref[...] = jnp.dot(h, w_ref[...],
                           preferred_element_type=jnp.float32) * dinv


def _tc_last_body(zp_ref, y_ref, dinv_ref, b_ref, out_ref):
    dinv = dinv_ref[...]
    out_ref[...] = dinv * (zp_ref[0] + zp_ref[1] + y_ref[...]) + b_ref[...]


def _row_spec(shape3=False):
    if shape3:
        return pl.BlockSpec((NC, BN, D), lambda i: (0, i, 0))
    return pl.BlockSpec((BN, D), lambda i: (i, 0))


_full_w = pl.BlockSpec((D, D), lambda i: (0, 0))
_full_b = pl.BlockSpec((1, D), lambda i: (0, 0))

_tc_first = pl.pallas_call(
    _tc_first_body,
    grid=(N // BN,),
    in_specs=[pl.BlockSpec((NC, BN, 16), lambda i: (0, i, 0)),
              _row_spec(), _full_w],
    out_specs=[_row_spec(), _row_spec()],
    out_shape=[jax.ShapeDtypeStruct((N, D), jnp.float32),
               jax.ShapeDtypeStruct((N, D), jnp.float32)],
)

_tc_mid = pl.pallas_call(
    _tc_mid_body,
    grid=(N // BN,),
    in_specs=[_row_spec(True), _row_spec(), _row_spec(), _full_b, _full_w],
    out_specs=_row_spec(),
    out_shape=jax.ShapeDtypeStruct((N, D), jnp.float32),
)

_tc_last = pl.pallas_call(
    _tc_last_body,
    grid=(N // BN,),
    in_specs=[_row_spec(True), _row_spec(), _row_spec(), _full_b],
    out_specs=_row_spec(),
    out_shape=jax.ShapeDtypeStruct((N, D), jnp.float32),
)


def kernel(x, edge_index, W1, b1, W2, b2, W3, b3):
    src = edge_index[0].astype(jnp.int32)
    dst = edge_index[1].astype(jnp.int32)
    degp = _sc_degree(dst)
    dinv_b, y = _tc_first(degp, x, W1)
    zp = _sc_scatter(y, src, dst)
    y = _tc_mid(zp, y, dinv_b, b1.reshape(1, D), W2)
    zp = _sc_scatter(y, src, dst)
    y = _tc_mid(zp, y, dinv_b, b2.reshape(1, D), W3)
    zp = _sc_scatter(y, src, dst)
    return _tc_last(zp, y, dinv_b, b3.reshape(1, D))
